# Initial kernel scaffold; baseline (speedup 1.0000x reference)
import jax
import jax.numpy as jnp
from jax.experimental import pallas as pl


def kernel(ch1, ch2, CP_locs, CP_idx):
    raise NotImplementedError("write your pallas kernel here")


# speedup vs baseline: 149.6055x; 149.6055x over previous
import functools

import jax
import jax.numpy as jnp
from jax import lax
from jax.experimental import pallas as pl
from jax.experimental.pallas import tpu as pltpu
from jax.experimental.pallas import tpu_sc as plsc

N = 524288
G = 202
TABW = G * G * 2
NC, NS, L = 2, 16, 16
NW = NC * NS
PER_W = N // NW
CHUNK = 2048
NCH = PER_W // CHUNK
VPC = CHUNK // L


def _combine(tab_v, fb, x, y, q11x, q11y):
    def weights(t):
        t2 = t * t
        w0 = ((-0.5 * t + 1.0) * t - 0.5) * t - 0.5
        w1 = t2 * (1.5 * t - 2.5)
        w2 = ((-1.5 * t + 2.0) * t - 0.5) * t + 0.5
        w3 = t2 * (0.5 * t - 0.5)
        return (w0, w1, w2, w3)

    wx = weights(x)
    wy = weights(y)
    outx = None
    outy = None
    for i in range(4):
        tx = None
        ty = None
        for j in range(4):
            if i == 1 and j == 1:
                qx, qy = q11x, q11y
            else:
                off = i * (2 * G) + j * 2
                qx = plsc.load_gather(tab_v, [fb + off])
                qy = plsc.load_gather(tab_v, [fb + (off + 1)])
            tx = wy[j] * qx if tx is None else tx + wy[j] * qx
            ty = wy[j] * qy if ty is None else ty + wy[j] * qy
        outx = wx[i] * tx if outx is None else outx + wx[i] * tx
        outy = wx[i] * ty if outy is None else outy + wx[i] * ty
    return outx, outy


def kernel(ch1, ch2, CP_locs, CP_idx):
    tab = CP_locs.reshape(-1)
    ch1f = ch1.reshape(-1)
    ch2f = ch2.reshape(-1)
    idxf = CP_idx.reshape(-1)

    mesh = plsc.VectorSubcoreMesh(core_axis_name="c", subcore_axis_name="s")

    @functools.partial(
        pl.kernel,
        mesh=mesh,
        out_type=jax.ShapeDtypeStruct((NW * L,), jnp.float32),
        compiler_params=pltpu.CompilerParams(needs_layout_passes=False),
        scratch_types=[
            pltpu.VMEM((TABW,), jnp.float32),
            pltpu.VMEM((CHUNK * 2,), jnp.int32),
            pltpu.VMEM((CHUNK * 2,), jnp.float32),
            pltpu.VMEM((CHUNK * 2,), jnp.float32),
            pltpu.VMEM((L,), jnp.float32),
        ],
    )
    def k(ch1_hbm, ch2_hbm, tab_hbm, idx_hbm, out_hbm,
          tab_v, idx_v, c1_v, c2_v, acc_v):
        wid = lax.axis_index("s") * NC + lax.axis_index("c")
        pltpu.sync_copy(tab_hbm, tab_v)
        e2 = lax.iota(jnp.int32, L) * 2

        def chunk_body(ci, acc):
            base = (wid * PER_W + ci * CHUNK) * 2
            pltpu.sync_copy(idx_hbm.at[pl.ds(base, CHUNK * 2)], idx_v)
            pltpu.sync_copy(ch1_hbm.at[pl.ds(base, CHUNK * 2)], c1_v)
            pltpu.sync_copy(ch2_hbm.at[pl.ds(base, CHUNK * 2)], c2_v)

            def vreg_body(v, acc):
                p = e2 + v * (2 * L)
                i0 = plsc.load_gather(idx_v, [p])
                i1 = plsc.load_gather(idx_v, [p + 1])
                fb = i0 * (2 * G) + i1 * 2 - (2 * G + 2)
                q11x = plsc.load_gather(tab_v, [fb + (2 * G + 2)])
                q11y = plsc.load_gather(tab_v, [fb + (2 * G + 3)])
                ch2x = plsc.load_gather(c2_v, [p])
                ch2y = plsc.load_gather(c2_v, [p + 1])
                x = ch2x - q11x
                y = ch2y - q11y
                outx, outy = _combine(tab_v, fb, x, y, q11x, q11y)
                dx = plsc.load_gather(c1_v, [p]) - outx
                dy = plsc.load_gather(c1_v, [p + 1]) - outy
                return acc + (dx * dx + dy * dy)

            return lax.fori_loop(0, VPC, vreg_body, acc)

        acc = lax.fori_loop(0, NCH, chunk_body, jnp.zeros((L,), jnp.float32))
        acc_v[...] = acc
        pltpu.sync_copy(acc_v, out_hbm.at[pl.ds(wid * L, L)])

    parts = k(ch1f, ch2f, tab, idxf)
    return jnp.sum(parts)


# speedup vs baseline: 1379.8356x; 9.2232x over previous
import functools

import jax
import jax.numpy as jnp
from jax import lax
from jax.experimental import pallas as pl
from jax.experimental.pallas import tpu as pltpu
from jax.experimental.pallas import tpu_sc as plsc

N = 524288
G = 202
TABW = G * G * 2
NC, NS, L = 2, 16, 16
NW = NC * NS
PER_W = N // NW
CHUNK = 2048
NCH = PER_W // CHUNK
VPC = CHUNK // L


def _weights(t):
    t2 = t * t
    w0 = ((-0.5 * t + 1.0) * t - 0.5) * t - 0.5
    w1 = t2 * (1.5 * t - 2.5)
    w2 = ((-1.5 * t + 2.0) * t - 0.5) * t + 0.5
    w3 = t2 * (0.5 * t - 0.5)
    return (w0, w1, w2, w3)


def _xy_flat(a):
    return a.reshape(N // 128, 128, 2).transpose(0, 2, 1).reshape(-1)


def kernel(ch1, ch2, CP_locs, CP_idx):
    tab = CP_locs.reshape(-1)
    ch1f = _xy_flat(ch1)
    ch2f = _xy_flat(ch2)
    idxf = _xy_flat(CP_idx)

    mesh = plsc.VectorSubcoreMesh(core_axis_name="c", subcore_axis_name="s")

    @functools.partial(
        pl.kernel,
        mesh=mesh,
        out_type=jax.ShapeDtypeStruct((NW * L,), jnp.float32),
        compiler_params=pltpu.CompilerParams(needs_layout_passes=False),
        scratch_types=[
            pltpu.VMEM((TABW,), jnp.float32),
            pltpu.VMEM((CHUNK * 2,), jnp.int32),
            pltpu.VMEM((CHUNK * 2,), jnp.float32),
            pltpu.VMEM((CHUNK * 2,), jnp.float32),
            pltpu.VMEM((L,), jnp.float32),
        ],
    )
    def k(ch1_hbm, ch2_hbm, tab_hbm, idx_hbm, out_hbm,
          tab_v, idx_v, c1_v, c2_v, acc_v):
        wid = lax.axis_index("s") * NC + lax.axis_index("c")
        pltpu.sync_copy(tab_hbm, tab_v)

        def chunk_body(ci, acc):
            base = (wid * PER_W + ci * CHUNK) * 2
            pltpu.sync_copy(idx_hbm.at[pl.ds(base, CHUNK * 2)], idx_v)
            pltpu.sync_copy(ch1_hbm.at[pl.ds(base, CHUNK * 2)], c1_v)
            pltpu.sync_copy(ch2_hbm.at[pl.ds(base, CHUNK * 2)], c2_v)

            def vreg_body(v, acc):
                xo = (v >> 3) * 256 + (v & 7) * 16
                yo = xo + 128
                i0 = idx_v[pl.ds(xo, L)]
                i1 = idx_v[pl.ds(yo, L)]
                fb = i0 * (2 * G) + i1 * 2 - (2 * G + 2)
                q11x = plsc.load_gather(tab_v, [fb + (2 * G + 2)])
                q11y = plsc.load_gather(tab_v, [fb + (2 * G + 3)])
                x = c2_v[pl.ds(xo, L)] - q11x
                y = c2_v[pl.ds(yo, L)] - q11y
                wx = _weights(x)
                wy = _weights(y)
                outx = None
                outy = None
                for i in range(4):
                    tx = None
                    ty = None
                    for j in range(4):
                        if i == 1 and j == 1:
                            qx, qy = q11x, q11y
                        else:
                            off = i * (2 * G) + j * 2
                            qx = plsc.load_gather(tab_v, [fb + off])
                            qy = plsc.load_gather(tab_v, [fb + (off + 1)])
                        tx = wy[j] * qx if tx is None else tx + wy[j] * qx
                        ty = wy[j] * qy if ty is None else ty + wy[j] * qy
                    outx = wx[i] * tx if outx is None else outx + wx[i] * tx
                    outy = wx[i] * ty if outy is None else outy + wx[i] * ty
                dx = c1_v[pl.ds(xo, L)] - outx
                dy = c1_v[pl.ds(yo, L)] - outy
                return acc + (dx * dx + dy * dy)

            return lax.fori_loop(0, VPC, vreg_body, acc)

        acc = lax.fori_loop(0, NCH, chunk_body, jnp.zeros((L,), jnp.float32))
        acc_v[...] = acc
        pltpu.sync_copy(acc_v, out_hbm.at[pl.ds(wid * L, L)])

    parts = k(ch1f, ch2f, tab, idxf)
    return jnp.sum(parts)
